# baseline (speedup 1.0000x reference)
import functools
import math

import jax
import jax.numpy as jnp
from jax import lax
from jax.experimental import pallas as pl
from jax.experimental.pallas import tpu as pltpu
from jax.experimental.pallas import tpu_sc as plsc

_M = 4
_COS_M = math.cos(_M)
_SIN_M = math.sin(_M)

_GW = 128


def _sc_gather(table, row_idx):
    n = row_idx.shape[0]
    info = plsc.get_sparse_core_info()
    nw = info.num_cores * info.num_subcores
    b_per_w = n // nw
    mesh = plsc.VectorSubcoreMesh(core_axis_name="c", subcore_axis_name="s")

    @functools.partial(
        pl.kernel,
        mesh=mesh,
        out_type=jax.ShapeDtypeStruct((n, _GW), jnp.float32),
        scratch_types=[
            pltpu.VMEM((b_per_w,), jnp.int32),
            pltpu.VMEM((b_per_w, _GW), jnp.float32),
            pltpu.SemaphoreType.DMA,
        ],
    )
    def gather_k(table_hbm, idx_hbm, out_hbm, idx_v, rows_v, sem):
        wid = lax.axis_index("s") * info.num_cores + lax.axis_index("c")
        base = wid * b_per_w
        pltpu.sync_copy(idx_hbm.at[pl.ds(base, b_per_w)], idx_v)
        pltpu.async_copy(table_hbm.at[idx_v], rows_v, sem).wait()
        pltpu.sync_copy(rows_v, out_hbm.at[pl.ds(base, b_per_w)])

    return gather_k(table, row_idx)


def _tc_body(x_ref, rows_ref, off_ref, out_ref, acc_ref, *, c):
    j = pl.program_id(0)
    nj = pl.num_programs(0)
    bc = x_ref.shape[1]

    @pl.when(j == 0)
    def _init():
        acc_ref[0, 0] = 0.0

    @pl.when(j < nj - 1)
    def _interior():
        x = x_ref[...]
        acc_ref[0, 0] += jnp.sum(x * x)

    @pl.when(j == nj - 1)
    def _last():
        x = x_ref[...]
        col = jax.lax.broadcasted_iota(jnp.int32, x.shape, 1)
        xm = jnp.where(col < c - j * bc, x, 0.0)
        acc_ref[0, 0] += jnp.sum(xm * xm)
        rows = rows_ref[...]
        lane = jax.lax.broadcasted_iota(jnp.int32, rows.shape, 1)
        v = jnp.sum(jnp.where(lane == off_ref[...], rows, 0.0), axis=1,
                    keepdims=True)
        phi = v * _COS_M - jnp.sqrt(jnp.maximum(1.0 - v * v, 0.0)) * _SIN_M
        corr = jnp.sum(phi * phi - v * v)
        total_n = rows_ref.shape[0] * c
        out_ref[0, 0] = (acc_ref[0, 0] + corr) * (_M * _M / total_n)


def kernel(input, label):
    b, c = input.shape
    lbl = label.astype(jnp.int32)
    p = jnp.arange(b, dtype=jnp.int32) * c + lbl
    rows = _sc_gather(input.reshape(b * c // _GW, _GW), p // _GW)
    off = (p % _GW).reshape(b, 1)

    bc = 2560
    grid = (pl.cdiv(c, bc),)
    out = pl.pallas_call(
        functools.partial(_tc_body, c=c),
        grid=grid,
        in_specs=[
            pl.BlockSpec((b, bc), lambda j: (0, j)),
            pl.BlockSpec((b, _GW), lambda j: (0, 0)),
            pl.BlockSpec((b, 1), lambda j: (0, 0)),
        ],
        out_specs=pl.BlockSpec(memory_space=pltpu.SMEM),
        out_shape=jax.ShapeDtypeStruct((1, 1), jnp.float32),
        scratch_shapes=[
            pltpu.SMEM((1, 1), jnp.float32),
        ],
    )(input, rows, off)
    return out.reshape(())


# speedup vs baseline: 1.0734x; 1.0734x over previous
import functools
import math

import jax
import jax.numpy as jnp
from jax import lax
from jax.experimental import pallas as pl
from jax.experimental.pallas import tpu as pltpu
from jax.experimental.pallas import tpu_sc as plsc

_M = 4
_COS_M = math.cos(_M)
_SIN_M = math.sin(_M)

_GW = 128


def _sc_gather(table, row_idx):
    n = row_idx.shape[0]
    info = plsc.get_sparse_core_info()
    nw = info.num_cores * info.num_subcores
    b_per_w = n // nw
    mesh = plsc.VectorSubcoreMesh(core_axis_name="c", subcore_axis_name="s")

    @functools.partial(
        pl.kernel,
        mesh=mesh,
        out_type=jax.ShapeDtypeStruct((n, _GW), jnp.float32),
        scratch_types=[
            pltpu.VMEM((b_per_w,), jnp.int32),
            pltpu.VMEM((b_per_w, _GW), jnp.float32),
            pltpu.SemaphoreType.DMA,
        ],
    )
    def gather_k(table_hbm, idx_hbm, out_hbm, idx_v, rows_v, sem):
        wid = lax.axis_index("s") * info.num_cores + lax.axis_index("c")
        base = wid * b_per_w
        pltpu.sync_copy(idx_hbm.at[pl.ds(base, b_per_w)], idx_v)
        pltpu.async_copy(table_hbm.at[idx_v], rows_v, sem).wait()
        pltpu.sync_copy(rows_v, out_hbm.at[pl.ds(base, b_per_w)])

    return gather_k(table, row_idx)


def _tc_body(x_ref, rows_ref, off_ref, out_ref, acc_ref, *, inv_n):
    j = pl.program_id(0)
    nj = pl.num_programs(0)

    @pl.when(j == 0)
    def _init():
        acc_ref[0, 0] = 0.0

    x = x_ref[...]
    acc_ref[0, 0] += jnp.sum(x * x)

    @pl.when(j == nj - 1)
    def _last():
        rows = rows_ref[...]
        lane = jax.lax.broadcasted_iota(jnp.int32, rows.shape, 1)
        v = jnp.sum(jnp.where(lane == off_ref[...], rows, 0.0), axis=1,
                    keepdims=True)
        phi = v * _COS_M - jnp.sqrt(jnp.maximum(1.0 - v * v, 0.0)) * _SIN_M
        corr = jnp.sum(phi * phi - v * v)
        out_ref[0, 0] = (acc_ref[0, 0] + corr) * (_M * _M * inv_n)


def kernel(input, label):
    b, c = input.shape
    lbl = label.astype(jnp.int32)
    p = jnp.arange(b, dtype=jnp.int32) * c + lbl
    flat = input.reshape(b * c // _GW, _GW)
    rows = _sc_gather(flat, p // _GW)
    off = (p % _GW).reshape(b, 1)

    br = 25000
    assert flat.shape[0] % br == 0
    grid = (flat.shape[0] // br,)
    out = pl.pallas_call(
        functools.partial(_tc_body, inv_n=1.0 / (b * c)),
        grid=grid,
        in_specs=[
            pl.BlockSpec((br, _GW), lambda j: (j, 0)),
            pl.BlockSpec((b, _GW), lambda j: (0, 0)),
            pl.BlockSpec((b, 1), lambda j: (0, 0)),
        ],
        out_specs=pl.BlockSpec(memory_space=pltpu.SMEM),
        out_shape=jax.ShapeDtypeStruct((1, 1), jnp.float32),
        scratch_shapes=[
            pltpu.SMEM((1, 1), jnp.float32),
        ],
    )(flat, rows, off)
    return out.reshape(())


# speedup vs baseline: 2.2106x; 2.0595x over previous
import functools
import math

import jax
import jax.numpy as jnp
from jax.experimental import pallas as pl
from jax.experimental.pallas import tpu as pltpu

_M = 4
_COS_M = math.cos(_M)
_SIN_M = math.sin(_M)


def _tc_body(x_ref, lbl_ref, out_ref, acc_ref, gacc_ref, *, c):
    j = pl.program_id(0)
    nj = pl.num_programs(0)
    br = x_ref.shape[0]

    @pl.when(j == 0)
    def _init():
        acc_ref[0, 0] = 0.0

    x = x_ref[...]
    acc_ref[0, 0] += jnp.sum(x * x)

    col = jax.lax.broadcasted_iota(jnp.int32, x.shape, 1)
    g = jnp.sum(jnp.where(col == lbl_ref[...], x, 0.0), axis=1, keepdims=True)
    gacc_ref[pl.ds(j * br, br), :] = g

    @pl.when(j == nj - 1)
    def _fin():
        v = gacc_ref[...]
        phi = v * _COS_M - jnp.sqrt(jnp.maximum(1.0 - v * v, 0.0)) * _SIN_M
        corr = jnp.sum(phi * phi - v * v)
        total_n = gacc_ref.shape[0] * c
        out_ref[0, 0, 0] = (acc_ref[0, 0] + corr) * (_M * _M / total_n)


def kernel(input, label):
    b, c = input.shape
    br = 64
    grid = (b // br,)
    lbl = label.astype(jnp.int32).reshape(b, 1)

    out = pl.pallas_call(
        functools.partial(_tc_body, c=c),
        grid=grid,
        in_specs=[
            pl.BlockSpec((br, c), lambda j: (j, 0)),
            pl.BlockSpec((br, 1), lambda j: (j, 0)),
        ],
        out_specs=pl.BlockSpec((1, 1, 1), lambda j: (0, 0, 0),
                               memory_space=pltpu.SMEM),
        out_shape=jax.ShapeDtypeStruct((1, 1, 1), jnp.float32),
        scratch_shapes=[
            pltpu.SMEM((1, 1), jnp.float32),
            pltpu.VMEM((b, 1), jnp.float32),
        ],
    )(input, lbl)
    return out.reshape(())
